# Initial kernel scaffold; baseline (speedup 1.0000x reference)
import jax
import jax.numpy as jnp
from jax.experimental import pallas as pl


def kernel(x, edge_index, W_pool0, b_pool0, W_self0, W_neigh0, bias0, W_pool1, b_pool1, W_self1, W_neigh1, bias1, W_pool2, b_pool2, W_self2, W_neigh2, bias2):
    raise NotImplementedError("write your pallas kernel here")


# speedup vs baseline: 1.4684x; 1.4684x over previous
import functools

import jax
import jax.numpy as jnp
from jax import lax
from jax.experimental import pallas as pl
from jax.experimental.pallas import tpu as pltpu
from jax.experimental.pallas import tpu_sc as plsc

NN = 10000
EE = 320000
NC, NS, LANES = 2, 16, 16
NW = NC * NS

EDGE_CHUNK = 4000


def _sc_segment_max(qT_flat, src, dst, D):
  Dw = D // NW
  n_chunks = EE // EDGE_CHUNK
  n_vec = EDGE_CHUNK // LANES

  mesh = plsc.VectorSubcoreMesh(
      core_axis_name="c", subcore_axis_name="s",
      num_cores=NC, num_subcores=NS)

  def body(q_hbm, src_hbm, dst_hbm, out_hbm, q_v, acc_v, s_v, d_v):
    wid = lax.axis_index("s") * NC + lax.axis_index("c")
    fbase = wid * (Dw * NN)

    pltpu.sync_copy(q_hbm.at[pl.ds(fbase, Dw * NN)], q_v)

    def zero_body(i, _):
      acc_v[pl.ds(i * LANES, LANES)] = jnp.zeros((LANES,), jnp.float32)
      return 0
    lax.fori_loop(0, (Dw * NN) // LANES, zero_body, 0)

    def chunk_body(c, _):
      pltpu.sync_copy(src_hbm.at[pl.ds(c * EDGE_CHUNK, EDGE_CHUNK)], s_v)
      pltpu.sync_copy(dst_hbm.at[pl.ds(c * EDGE_CHUNK, EDGE_CHUNK)], d_v)

      def vec_body(j, _):
        s = s_v[pl.ds(j * LANES, LANES)]
        d = d_v[pl.ds(j * LANES, LANES)]
        vals = [plsc.load_gather(q_v, [s + f * NN]) for f in range(Dw)]
        idxs = [d + f * NN for f in range(Dw)]

        def upd_cond(carry):
          pend, _ = carry
          return jnp.any(pend)

        def upd_body(carry):
          pend, it = carry
          alive = jnp.zeros((LANES,), jnp.bool_)
          for f in range(Dw):
            old = plsc.load_gather(acc_v, [idxs[f]])
            new = jnp.maximum(old, vals[f])
            plsc.store_scatter(acc_v, [idxs[f]], new, mask=pend)
            rb = plsc.load_gather(acc_v, [idxs[f]])
            alive = jnp.logical_or(alive, rb < vals[f])
          return jnp.logical_and(pend, alive), it + 1

        pend0 = jnp.ones((LANES,), jnp.bool_)
        lax.while_loop(upd_cond, upd_body, (pend0, jnp.int32(0)))
        return 0

      lax.fori_loop(0, n_vec, vec_body, 0)
      return 0

    lax.fori_loop(0, n_chunks, chunk_body, 0)

    pltpu.sync_copy(acc_v, out_hbm.at[pl.ds(fbase, Dw * NN)])

  run = pl.kernel(
      body,
      out_type=jax.ShapeDtypeStruct((D * NN,), jnp.float32),
      mesh=mesh,
      compiler_params=pltpu.CompilerParams(
          needs_layout_passes=False, use_tc_tiling_on_sc=False),
      scratch_types=[
          pltpu.VMEM((Dw * NN,), jnp.float32),
          pltpu.VMEM((Dw * NN,), jnp.float32),
          pltpu.VMEM((EDGE_CHUNK,), jnp.int32),
          pltpu.VMEM((EDGE_CHUNK,), jnp.int32),
      ],
  )
  return run(qT_flat, src, dst)


def _tc_pool_body(x_ref, w_ref, b_ref, o_ref):
  q = lax.dot_general(w_ref[...], x_ref[...], (((0,), (1,)), ((), ())),
                      preferred_element_type=jnp.float32)
  o_ref[...] = jnp.maximum(q + b_ref[...], 0.0)


def _tc_pool(x, W_pool, b_pool):
  D = W_pool.shape[1]
  out = pl.pallas_call(
      _tc_pool_body,
      out_shape=jax.ShapeDtypeStruct((D, NN), jnp.float32),
  )(x, W_pool, b_pool[:, None])
  return out


def _tc_mid_body(h_ref, nT_ref, ws_ref, wn_ref, bias_ref, wp_ref, bp_ref,
                 h_out_ref, qT_out_ref):
  hs = lax.dot_general(h_ref[...], ws_ref[...], (((1,), (0,)), ((), ())),
                       preferred_element_type=jnp.float32)
  hn = lax.dot_general(nT_ref[...], wn_ref[...], (((0,), (0,)), ((), ())),
                       preferred_element_type=jnp.float32)
  h_new = jnp.maximum(hs + hn + bias_ref[...], 0.0)
  h_out_ref[...] = h_new
  q = lax.dot_general(wp_ref[...], h_new, (((0,), (1,)), ((), ())),
                      preferred_element_type=jnp.float32)
  qT_out_ref[...] = jnp.maximum(q + bp_ref[...], 0.0)


def _tc_mid(h, neighT, W_self, W_neigh, bias, W_pool, b_pool):
  dout = W_self.shape[1]
  dq = W_pool.shape[1]
  h_new, qT = pl.pallas_call(
      _tc_mid_body,
      out_shape=(jax.ShapeDtypeStruct((NN, dout), jnp.float32),
                 jax.ShapeDtypeStruct((dq, NN), jnp.float32)),
  )(h, neighT, W_self, W_neigh, bias[None, :], W_pool, b_pool[:, None])
  return h_new, qT


def _tc_final_body(h_ref, nT_ref, ws_ref, wn_ref, bias_ref, o_ref):
  hs = lax.dot_general(h_ref[...], ws_ref[...], (((1,), (0,)), ((), ())),
                       preferred_element_type=jnp.float32)
  hn = lax.dot_general(nT_ref[...], wn_ref[...], (((0,), (0,)), ((), ())),
                       preferred_element_type=jnp.float32)
  z = hs + hn + bias_ref[...]
  o_ref[...] = 1.0 / (1.0 + jnp.exp(-z))


def _tc_final(h, neighT, W_self, W_neigh, bias):
  dout = W_self.shape[1]
  out = pl.pallas_call(
      _tc_final_body,
      out_shape=jax.ShapeDtypeStruct((NN, dout), jnp.float32),
  )(h, neighT, W_self, W_neigh, bias[None, :])
  return out


def kernel(x, edge_index,
           W_pool0, b_pool0, W_self0, W_neigh0, bias0,
           W_pool1, b_pool1, W_self1, W_neigh1, bias1,
           W_pool2, b_pool2, W_self2, W_neigh2, bias2):
  src = edge_index[0].astype(jnp.int32)
  dst = edge_index[1].astype(jnp.int32)

  q0T = _tc_pool(x, W_pool0, b_pool0)
  n0T = _sc_segment_max(q0T.reshape(-1), src, dst, 128)
  h1, q1T = _tc_mid(x, n0T.reshape(128, NN),
                    W_self0, W_neigh0, bias0, W_pool1, b_pool1)
  n1T = _sc_segment_max(q1T.reshape(-1), src, dst, 32)
  h2, q2T = _tc_mid(h1, n1T.reshape(32, NN),
                    W_self1, W_neigh1, bias1, W_pool2, b_pool2)
  n2T = _sc_segment_max(q2T.reshape(-1), src, dst, 32)
  out = _tc_final(h2, n2T.reshape(32, NN), W_self2, W_neigh2, bias2)
  return out


# speedup vs baseline: 1.8133x; 1.2349x over previous
import functools

import jax
import jax.numpy as jnp
from jax import lax
from jax.experimental import pallas as pl
from jax.experimental.pallas import tpu as pltpu
from jax.experimental.pallas import tpu_sc as plsc

NN = 10000
EE = 320000
NC, NS, LANES = 2, 16, 16
NW = NC * NS

EDGE_CHUNK = 4000


def _sc_segment_max(qT_flat, src, dst, D):
  Dw = D // NW
  n_chunks = EE // EDGE_CHUNK
  n_vec = EDGE_CHUNK // LANES

  mesh = plsc.VectorSubcoreMesh(
      core_axis_name="c", subcore_axis_name="s",
      num_cores=NC, num_subcores=NS)

  def body(q_hbm, src_hbm, dst_hbm, out_hbm, q_v, acc_v, s_v, d_v, chk_v,
           sems):
    wid = lax.axis_index("s") * NC + lax.axis_index("c")
    fbase = wid * (Dw * NN)

    pltpu.sync_copy(q_hbm.at[pl.ds(fbase, Dw * NN)], q_v)

    def zero_body(i, _):
      acc_v[pl.ds(i * LANES, LANES)] = jnp.zeros((LANES,), jnp.float32)
      return 0
    lax.fori_loop(0, (Dw * NN) // LANES, zero_body, 0, unroll=4)

    lane_id = lax.iota(jnp.int32, LANES)

    def start_fetch(c, slot):
      pltpu.make_async_copy(
          src_hbm.at[pl.ds(c * EDGE_CHUNK, EDGE_CHUNK)],
          s_v.at[pl.ds(slot * EDGE_CHUNK, EDGE_CHUNK)], sems.at[slot]).start()
      pltpu.make_async_copy(
          dst_hbm.at[pl.ds(c * EDGE_CHUNK, EDGE_CHUNK)],
          d_v.at[pl.ds(slot * EDGE_CHUNK, EDGE_CHUNK)], sems.at[slot]).start()

    def wait_fetch(slot):
      pltpu.make_async_copy(
          src_hbm.at[pl.ds(0, EDGE_CHUNK)],
          s_v.at[pl.ds(slot * EDGE_CHUNK, EDGE_CHUNK)], sems.at[slot]).wait()
      pltpu.make_async_copy(
          dst_hbm.at[pl.ds(0, EDGE_CHUNK)],
          d_v.at[pl.ds(slot * EDGE_CHUNK, EDGE_CHUNK)], sems.at[slot]).wait()

    start_fetch(0, 0)

    def chunk_body(c, _):
      slot = lax.rem(c, 2)
      pl.when(c + 1 < n_chunks)(lambda: start_fetch(c + 1, 1 - slot))
      wait_fetch(slot)
      ebase = slot * EDGE_CHUNK

      def vec_body(j, _):
        s = s_v[pl.ds(ebase + j * LANES, LANES)]
        d = d_v[pl.ds(ebase + j * LANES, LANES)]
        vals = [plsc.load_gather(q_v, [s if f == 0 else s + f * NN])
                for f in range(Dw)]
        idxs = [d if f == 0 else d + f * NN for f in range(Dw)]

        plsc.store_scatter(chk_v, [d], lane_id)
        uniq = plsc.load_gather(chk_v, [d]) == lane_id
        no_dup = jnp.all(uniq)

        def fast_path():
          for f in range(Dw):
            old = plsc.load_gather(acc_v, [idxs[f]])
            plsc.store_scatter(acc_v, [idxs[f]], jnp.maximum(old, vals[f]))

        def slow_path():
          def upd_cond(carry):
            return jnp.any(carry)

          def upd_body(pend):
            alive = jnp.zeros((LANES,), jnp.bool_)
            for f in range(Dw):
              old = plsc.load_gather(acc_v, [idxs[f]])
              new = jnp.maximum(old, vals[f])
              plsc.store_scatter(acc_v, [idxs[f]], new, mask=pend)
              rb = plsc.load_gather(acc_v, [idxs[f]])
              alive = jnp.logical_or(alive, rb < vals[f])
            return jnp.logical_and(pend, alive)

          lax.while_loop(upd_cond, upd_body, jnp.ones((LANES,), jnp.bool_))

        lax.cond(no_dup, fast_path, slow_path)
        return 0

      lax.fori_loop(0, n_vec, vec_body, 0, unroll=2)
      return 0

    lax.fori_loop(0, n_chunks, chunk_body, 0)

    pltpu.sync_copy(acc_v, out_hbm.at[pl.ds(fbase, Dw * NN)])

  run = pl.kernel(
      body,
      out_type=jax.ShapeDtypeStruct((D * NN,), jnp.float32),
      mesh=mesh,
      compiler_params=pltpu.CompilerParams(
          needs_layout_passes=False, use_tc_tiling_on_sc=False,
          disable_bounds_checks=True),
      scratch_types=[
          pltpu.VMEM((Dw * NN,), jnp.float32),
          pltpu.VMEM((Dw * NN,), jnp.float32),
          pltpu.VMEM((2 * EDGE_CHUNK,), jnp.int32),
          pltpu.VMEM((2 * EDGE_CHUNK,), jnp.int32),
          pltpu.VMEM((NN,), jnp.int32),
          pltpu.SemaphoreType.DMA((2,)),
      ],
  )
  return run(qT_flat, src, dst)


def _tc_pool_body(x_ref, w_ref, b_ref, o_ref):
  q = lax.dot_general(w_ref[...], x_ref[...], (((0,), (1,)), ((), ())),
                      preferred_element_type=jnp.float32)
  o_ref[...] = jnp.maximum(q + b_ref[...], 0.0)


def _tc_pool(x, W_pool, b_pool):
  D = W_pool.shape[1]
  out = pl.pallas_call(
      _tc_pool_body,
      out_shape=jax.ShapeDtypeStruct((D, NN), jnp.float32),
  )(x, W_pool, b_pool[:, None])
  return out


def _tc_mid_body(h_ref, nT_ref, ws_ref, wn_ref, bias_ref, wp_ref, bp_ref,
                 h_out_ref, qT_out_ref):
  hs = lax.dot_general(h_ref[...], ws_ref[...], (((1,), (0,)), ((), ())),
                       preferred_element_type=jnp.float32)
  hn = lax.dot_general(nT_ref[...], wn_ref[...], (((0,), (0,)), ((), ())),
                       preferred_element_type=jnp.float32)
  h_new = jnp.maximum(hs + hn + bias_ref[...], 0.0)
  h_out_ref[...] = h_new
  q = lax.dot_general(wp_ref[...], h_new, (((0,), (1,)), ((), ())),
                      preferred_element_type=jnp.float32)
  qT_out_ref[...] = jnp.maximum(q + bp_ref[...], 0.0)


def _tc_mid(h, neighT, W_self, W_neigh, bias, W_pool, b_pool):
  dout = W_self.shape[1]
  dq = W_pool.shape[1]
  h_new, qT = pl.pallas_call(
      _tc_mid_body,
      out_shape=(jax.ShapeDtypeStruct((NN, dout), jnp.float32),
                 jax.ShapeDtypeStruct((dq, NN), jnp.float32)),
  )(h, neighT, W_self, W_neigh, bias[None, :], W_pool, b_pool[:, None])
  return h_new, qT


def _tc_final_body(h_ref, nT_ref, ws_ref, wn_ref, bias_ref, o_ref):
  hs = lax.dot_general(h_ref[...], ws_ref[...], (((1,), (0,)), ((), ())),
                       preferred_element_type=jnp.float32)
  hn = lax.dot_general(nT_ref[...], wn_ref[...], (((0,), (0,)), ((), ())),
                       preferred_element_type=jnp.float32)
  z = hs + hn + bias_ref[...]
  o_ref[...] = 1.0 / (1.0 + jnp.exp(-z))


def _tc_final(h, neighT, W_self, W_neigh, bias):
  dout = W_self.shape[1]
  out = pl.pallas_call(
      _tc_final_body,
      out_shape=jax.ShapeDtypeStruct((NN, dout), jnp.float32),
  )(h, neighT, W_self, W_neigh, bias[None, :])
  return out


def kernel(x, edge_index,
           W_pool0, b_pool0, W_self0, W_neigh0, bias0,
           W_pool1, b_pool1, W_self1, W_neigh1, bias1,
           W_pool2, b_pool2, W_self2, W_neigh2, bias2):
  src = edge_index[0].astype(jnp.int32)
  dst = edge_index[1].astype(jnp.int32)

  q0T = _tc_pool(x, W_pool0, b_pool0)
  n0T = _sc_segment_max(q0T.reshape(-1), src, dst, 128)
  h1, q1T = _tc_mid(x, n0T.reshape(128, NN),
                    W_self0, W_neigh0, bias0, W_pool1, b_pool1)
  n1T = _sc_segment_max(q1T.reshape(-1), src, dst, 32)
  h2, q2T = _tc_mid(h1, n1T.reshape(32, NN),
                    W_self1, W_neigh1, bias1, W_pool2, b_pool2)
  n2T = _sc_segment_max(q2T.reshape(-1), src, dst, 32)
  out = _tc_final(h2, n2T.reshape(32, NN), W_self2, W_neigh2, bias2)
  return out
